# baseline (speedup 1.0000x reference)
import functools

import jax
import jax.numpy as jnp
from jax.experimental import pallas as pl

NB_CLASSES = 12
NUM_POS = 768
SFD = NUM_POS // NB_CLASSES
N_NEG = 4
B, D, M = 32, 256, 8192

B_BLK = 2
M_BLK = B_BLK * SFD
GRID = B // B_BLK


def _perm_matrix(n_rows, inner):
    rows = jax.lax.broadcasted_iota(jnp.int32, (n_rows, n_rows), 0)
    cols = jax.lax.broadcasted_iota(jnp.int32, (n_rows, n_rows), 1)
    outer = n_rows // inner
    want = (rows % inner) * outer + rows // inner
    return (cols == want).astype(jnp.bfloat16)


def _body(x_ref, lbl_ref, y_ref, mem_ref, sim_ref, nsim_ref, oht_ref, yo_ref):
    mem = mem_ref[...].astype(jnp.bfloat16)
    xb = x_ref[...].astype(jnp.bfloat16)
    t_s = xb[:SFD].reshape(SFD * B_BLK, D)
    t = jax.lax.dot_general(
        _perm_matrix(SFD * B_BLK, SFD), t_s, (((1,), (0,)), ((), ())),
        preferred_element_type=jnp.float32).astype(jnp.bfloat16)
    sim_ref[...] = jax.lax.dot_general(
        t, mem, (((1,), (1,)), ((), ())),
        preferred_element_type=jnp.float32).reshape(B_BLK, SFD, M)
    n_s = xb[SFD:].reshape(N_NEG * B_BLK, D)
    noise = jax.lax.dot_general(
        _perm_matrix(N_NEG * B_BLK, N_NEG), n_s, (((1,), (0,)), ((), ())),
        preferred_element_type=jnp.float32).astype(jnp.bfloat16)
    nsim_ref[...] = jax.lax.dot_general(
        noise, mem[:NUM_POS], (((1,), (1,)), ((), ())),
        preferred_element_type=jnp.float32).reshape(B_BLK, N_NEG, NUM_POS)

    @pl.when(pl.program_id(0) == 0)
    def _():
        lbl = lbl_ref[...]
        classes = jax.lax.broadcasted_iota(jnp.int32, (NB_CLASSES, B), 0)
        eq_t = (lbl[None, :] == classes).astype(jnp.float32)
        cnt = jnp.sum(eq_t, axis=1, keepdims=True)
        oht_ref[...] = eq_t / jnp.maximum(cnt, 1.0)
        yo_ref[...] = y_ref[...]


@jax.jit
def kernel(x, y, visible, img_label, memory):
    xt = jnp.transpose(x, (1, 0, 2)).reshape(SFD + N_NEG, GRID, B_BLK, D)

    sim, nsim, oht, yo = pl.pallas_call(
        _body,
        grid=(GRID,),
        in_specs=[
            pl.BlockSpec((SFD + N_NEG, None, B_BLK, D), lambda i: (0, i, 0, 0)),
            pl.BlockSpec((B,), lambda i: (0,)),
            pl.BlockSpec((B, SFD), lambda i: (0, 0)),
            pl.BlockSpec((M, D), lambda i: (0, 0)),
        ],
        out_specs=[
            pl.BlockSpec((B_BLK, SFD, M), lambda i: (i, 0, 0)),
            pl.BlockSpec((B_BLK, N_NEG, NUM_POS), lambda i: (i, 0, 0)),
            pl.BlockSpec((NB_CLASSES, B), lambda i: (0, 0)),
            pl.BlockSpec((B, SFD), lambda i: (0, 0)),
        ],
        out_shape=[
            jax.ShapeDtypeStruct((B, SFD, M), jnp.float32),
            jax.ShapeDtypeStruct((B, N_NEG, NUM_POS), jnp.float32),
            jax.ShapeDtypeStruct((NB_CLASSES, B), jnp.float32),
            jax.ShapeDtypeStruct((B, SFD), y.dtype),
        ],
    )(xt, img_label, y, memory)

    return (sim, yo, nsim, oht.T)


# speedup vs baseline: 1.4056x; 1.4056x over previous
import functools

import jax
import jax.numpy as jnp
from jax.experimental import pallas as pl

NB_CLASSES = 12
NUM_POS = 768
SFD = NUM_POS // NB_CLASSES
N_NEG = 4
B, D, M = 32, 256, 8192

B_BLK = 8
M_BLK = B_BLK * SFD
GRID = B // B_BLK


def _perm_matrix(n_rows, inner):
    rows = jax.lax.broadcasted_iota(jnp.int32, (n_rows, n_rows), 0)
    cols = jax.lax.broadcasted_iota(jnp.int32, (n_rows, n_rows), 1)
    outer = n_rows // inner
    want = (rows % inner) * outer + rows // inner
    return (cols == want).astype(jnp.bfloat16)


def _body(x_ref, lbl_ref, y_ref, mem_ref, sim_ref, nsim_ref, oht_ref, yo_ref):
    mem = mem_ref[...].astype(jnp.bfloat16)
    xb = x_ref[...].astype(jnp.bfloat16)
    t_s = xb[:SFD].reshape(SFD * B_BLK, D)
    t = jax.lax.dot_general(
        _perm_matrix(SFD * B_BLK, SFD), t_s, (((1,), (0,)), ((), ())),
        preferred_element_type=jnp.float32).astype(jnp.bfloat16)
    sim_ref[...] = jax.lax.dot_general(
        t, mem, (((1,), (1,)), ((), ())),
        preferred_element_type=jnp.float32).reshape(B_BLK, SFD, M)
    n_s = xb[SFD:].reshape(N_NEG * B_BLK, D)
    noise = jax.lax.dot_general(
        _perm_matrix(N_NEG * B_BLK, N_NEG), n_s, (((1,), (0,)), ((), ())),
        preferred_element_type=jnp.float32).astype(jnp.bfloat16)
    nsim_ref[...] = jax.lax.dot_general(
        noise, mem[:NUM_POS], (((1,), (1,)), ((), ())),
        preferred_element_type=jnp.float32).reshape(B_BLK, N_NEG, NUM_POS)

    @pl.when(pl.program_id(0) == 0)
    def _():
        lbl = lbl_ref[...]
        classes = jax.lax.broadcasted_iota(jnp.int32, (NB_CLASSES, B), 0)
        eq_t = (lbl[None, :] == classes).astype(jnp.float32)
        cnt = jnp.sum(eq_t, axis=1, keepdims=True)
        oht_ref[...] = eq_t / jnp.maximum(cnt, 1.0)
        yo_ref[...] = y_ref[...]


@jax.jit
def kernel(x, y, visible, img_label, memory):
    xt = jnp.transpose(x, (1, 0, 2)).reshape(SFD + N_NEG, GRID, B_BLK, D)

    sim, nsim, oht, yo = pl.pallas_call(
        _body,
        grid=(GRID,),
        in_specs=[
            pl.BlockSpec((SFD + N_NEG, None, B_BLK, D), lambda i: (0, i, 0, 0)),
            pl.BlockSpec((B,), lambda i: (0,)),
            pl.BlockSpec((B, SFD), lambda i: (0, 0)),
            pl.BlockSpec((M, D), lambda i: (0, 0)),
        ],
        out_specs=[
            pl.BlockSpec((B_BLK, SFD, M), lambda i: (i, 0, 0)),
            pl.BlockSpec((B_BLK, N_NEG, NUM_POS), lambda i: (i, 0, 0)),
            pl.BlockSpec((NB_CLASSES, B), lambda i: (0, 0)),
            pl.BlockSpec((B, SFD), lambda i: (0, 0)),
        ],
        out_shape=[
            jax.ShapeDtypeStruct((B, SFD, M), jnp.float32),
            jax.ShapeDtypeStruct((B, N_NEG, NUM_POS), jnp.float32),
            jax.ShapeDtypeStruct((NB_CLASSES, B), jnp.float32),
            jax.ShapeDtypeStruct((B, SFD), y.dtype),
        ],
    )(xt, img_label, y, memory)

    return (sim, yo, nsim, oht.T)


# speedup vs baseline: 1.4097x; 1.0029x over previous
import functools

import jax
import jax.numpy as jnp
from jax.experimental import pallas as pl
from jax.experimental.pallas import tpu as pltpu

NB_CLASSES = 12
NUM_POS = 768
SFD = NUM_POS // NB_CLASSES
N_NEG = 4
B, D, M = 32, 256, 8192

B_BLK = 8
M_BLK = B_BLK * SFD
GRID = B // B_BLK


def _perm_matrix(n_rows, inner):
    rows = jax.lax.broadcasted_iota(jnp.int32, (n_rows, n_rows), 0)
    cols = jax.lax.broadcasted_iota(jnp.int32, (n_rows, n_rows), 1)
    outer = n_rows // inner
    want = (rows % inner) * outer + rows // inner
    return (cols == want).astype(jnp.bfloat16)


def _body(x_ref, lbl_ref, y_ref, mem_ref, sim_ref, nsim_ref, oht_ref, yo_ref):
    mem = mem_ref[...].astype(jnp.bfloat16)
    xb = x_ref[...].astype(jnp.bfloat16)
    t_s = xb[:SFD].reshape(SFD * B_BLK, D)
    t = jax.lax.dot_general(
        _perm_matrix(SFD * B_BLK, SFD), t_s, (((1,), (0,)), ((), ())),
        preferred_element_type=jnp.float32).astype(jnp.bfloat16)
    sim_ref[...] = jax.lax.dot_general(
        t, mem, (((1,), (1,)), ((), ())),
        preferred_element_type=jnp.float32).reshape(B_BLK, SFD, M)
    n_s = xb[SFD:].reshape(N_NEG * B_BLK, D)
    noise = jax.lax.dot_general(
        _perm_matrix(N_NEG * B_BLK, N_NEG), n_s, (((1,), (0,)), ((), ())),
        preferred_element_type=jnp.float32).astype(jnp.bfloat16)
    nsim_ref[...] = jax.lax.dot_general(
        noise, mem[:NUM_POS], (((1,), (1,)), ((), ())),
        preferred_element_type=jnp.float32).reshape(B_BLK, N_NEG, NUM_POS)

    @pl.when(pl.program_id(0) == 0)
    def _():
        lbl = lbl_ref[...]
        classes = jax.lax.broadcasted_iota(jnp.int32, (NB_CLASSES, B), 0)
        eq_t = (lbl[None, :] == classes).astype(jnp.float32)
        cnt = jnp.sum(eq_t, axis=1, keepdims=True)
        oht_ref[...] = eq_t / jnp.maximum(cnt, 1.0)
        yo_ref[...] = y_ref[...]


@jax.jit
def kernel(x, y, visible, img_label, memory):
    xt = jnp.transpose(x, (1, 0, 2)).reshape(SFD + N_NEG, GRID, B_BLK, D)

    sim, nsim, oht, yo = pl.pallas_call(
        _body,
        grid=(GRID,),
        in_specs=[
            pl.BlockSpec((SFD + N_NEG, None, B_BLK, D), lambda i: (0, i, 0, 0)),
            pl.BlockSpec((B,), lambda i: (0,)),
            pl.BlockSpec((B, SFD), lambda i: (0, 0)),
            pl.BlockSpec((M, D), lambda i: (0, 0)),
        ],
        out_specs=[
            pl.BlockSpec((B_BLK, SFD, M), lambda i: (i, 0, 0)),
            pl.BlockSpec((B_BLK, N_NEG, NUM_POS), lambda i: (i, 0, 0)),
            pl.BlockSpec((NB_CLASSES, B), lambda i: (0, 0)),
            pl.BlockSpec((B, SFD), lambda i: (0, 0)),
        ],
        out_shape=[
            jax.ShapeDtypeStruct((B, SFD, M), jnp.float32),
            jax.ShapeDtypeStruct((B, N_NEG, NUM_POS), jnp.float32),
            jax.ShapeDtypeStruct((NB_CLASSES, B), jnp.float32),
            jax.ShapeDtypeStruct((B, SFD), y.dtype),
        ],
        compiler_params=pltpu.CompilerParams(
            dimension_semantics=("parallel",)),
    )(xt, img_label, y, memory)

    return (sim, yo, nsim, oht.T)
